# Initial kernel scaffold; baseline (speedup 1.0000x reference)
import jax
import jax.numpy as jnp
from jax.experimental import pallas as pl


def kernel(x, W):
    raise NotImplementedError("write your pallas kernel here")


# speedup vs baseline: 2.4100x; 2.4100x over previous
import functools

import jax
import jax.numpy as jnp
from jax import lax
from jax.experimental import pallas as pl
from jax.experimental.pallas import tpu as pltpu
from jax.experimental.pallas import tpu_sc as plsc

NUM_ROWS = 400
EMB = 32
FEAT = 16
OUT_D = 78
LANES = 16

NC = 2
NS = 16
NW = NC * NS

CHUNK = 512


def _sc_embed(x_flat, w_flat, *, num_tokens):
    tok_per_w = num_tokens // NW
    n_chunks = tok_per_w // CHUNK

    mesh = plsc.VectorSubcoreMesh(core_axis_name="c", subcore_axis_name="s")

    @functools.partial(
        pl.kernel,
        out_type=jax.ShapeDtypeStruct((num_tokens * OUT_D,), jnp.float32),
        mesh=mesh,
        scratch_types=[
            pltpu.VMEM((NUM_ROWS * EMB,), jnp.float32),
            pltpu.VMEM((CHUNK * FEAT,), jnp.float32),
            pltpu.VMEM((CHUNK * OUT_D,), jnp.float32),
        ],
        compiler_params=pltpu.CompilerParams(needs_layout_passes=False),
    )
    def k(x_hbm, w_hbm, out_hbm, w_v, x_v, out_v):
        wid = lax.axis_index("s") * NC + lax.axis_index("c")
        base_tok = wid * tok_per_w

        pltpu.sync_copy(w_hbm, w_v)

        def chunk_body(ci, _):
            tb = base_tok + ci * CHUNK
            pltpu.sync_copy(x_hbm.at[pl.ds(tb * FEAT, CHUNK * FEAT)], x_v)

            def group_body(g, _):
                tvec = lax.iota(jnp.int32, LANES) + g * LANES
                sbase = tvec * FEAT
                obase = tvec * OUT_D

                fa = plsc.load_gather(x_v, [sbase])
                fo = plsc.load_gather(x_v, [sbase + 8])
                pa = jnp.clip(fa, 0.0, float(NUM_ROWS - 1)).astype(jnp.int32) * EMB
                po = jnp.clip(fo, 0.0, float(NUM_ROWS - 1)).astype(jnp.int32) * EMB

                for c in range(EMB):
                    va = plsc.load_gather(w_v, [pa + c])
                    plsc.store_scatter(out_v, [obase + c], va)
                    vo = plsc.load_gather(w_v, [po + c])
                    plsc.store_scatter(out_v, [obase + (EMB + 7 + c)], vo)
                for j in range(7):
                    vs = plsc.load_gather(x_v, [sbase + (1 + j)])
                    plsc.store_scatter(out_v, [obase + (EMB + j)], vs)
                    vs2 = plsc.load_gather(x_v, [sbase + (9 + j)])
                    plsc.store_scatter(out_v, [obase + (2 * EMB + 7 + j)], vs2)
                return 0

            lax.fori_loop(0, CHUNK // LANES, group_body, 0)
            pltpu.sync_copy(out_v, out_hbm.at[pl.ds(tb * OUT_D, CHUNK * OUT_D)])
            return 0

        lax.fori_loop(0, n_chunks, chunk_body, 0)

    return k(x_flat, w_flat)


def kernel(x, W):
    b, t, f = x.shape
    num_tokens = b * t
    out_flat = _sc_embed(x.reshape(-1), W.reshape(-1), num_tokens=num_tokens)
    return out_flat.reshape(b, t, OUT_D)


# speedup vs baseline: 3.4018x; 1.4115x over previous
import functools

import jax
import jax.numpy as jnp
from jax import lax
from jax.experimental import pallas as pl
from jax.experimental.pallas import tpu as pltpu
from jax.experimental.pallas import tpu_sc as plsc

NUM_ROWS = 400
EMB = 32
FEAT = 16
OUT_D = 78
LANES = 16

WSTRIDE = EMB + 1
XSTRIDE = FEAT + 1

NC = 2
NS = 16
NW = NC * NS

CHUNK = 512


def _sc_embed(x_flat, w_flat, *, num_tokens):
    tok_per_w = num_tokens // NW
    n_chunks = tok_per_w // CHUNK

    mesh = plsc.VectorSubcoreMesh(core_axis_name="c", subcore_axis_name="s")

    @functools.partial(
        pl.kernel,
        out_type=jax.ShapeDtypeStruct((num_tokens * OUT_D,), jnp.float32),
        mesh=mesh,
        scratch_types=[
            pltpu.VMEM((NUM_ROWS * EMB,), jnp.float32),
            pltpu.VMEM((NUM_ROWS * WSTRIDE,), jnp.float32),
            pltpu.VMEM((CHUNK * FEAT,), jnp.float32),
            pltpu.VMEM((CHUNK * XSTRIDE,), jnp.float32),
            pltpu.VMEM((CHUNK * OUT_D,), jnp.float32),
        ],
        compiler_params=pltpu.CompilerParams(needs_layout_passes=False),
    )
    def k(x_hbm, w_hbm, out_hbm, wraw_v, w_v, xraw_v, x_v, out_v):
        wid = lax.axis_index("s") * NC + lax.axis_index("c")
        base_tok = wid * tok_per_w
        iota = lax.iota(jnp.int32, LANES)

        pltpu.sync_copy(w_hbm, wraw_v)

        def wpack_body(i, _):
            src = wraw_v[pl.ds(i * LANES, LANES)]
            w_v[pl.ds(i * LANES + i // 2, LANES)] = src
            return 0

        lax.fori_loop(0, NUM_ROWS * EMB // LANES, wpack_body, 0)

        def chunk_body(ci, _):
            tb = base_tok + ci * CHUNK
            pltpu.sync_copy(x_hbm.at[pl.ds(tb * FEAT, CHUNK * FEAT)], xraw_v)

            def xpack_body(t, _):
                src = xraw_v[pl.ds(t * FEAT, LANES)]
                x_v[pl.ds(t * XSTRIDE, LANES)] = src
                return 0

            lax.fori_loop(0, CHUNK, xpack_body, 0)

            def group_body(g, _):
                tvec = iota + g * LANES
                sbase = tvec * XSTRIDE
                obase = tvec * OUT_D

                fa = plsc.load_gather(x_v, [sbase])
                fo = plsc.load_gather(x_v, [sbase + 8])
                ia = jnp.clip(fa, 0.0, float(NUM_ROWS - 1)).astype(jnp.int32)
                io = jnp.clip(fo, 0.0, float(NUM_ROWS - 1)).astype(jnp.int32)
                pa = ia * WSTRIDE
                po = io * WSTRIDE

                for c in range(EMB):
                    va = plsc.load_gather(w_v, [pa + c])
                    plsc.store_scatter(out_v, [obase + c], va)
                    vo = plsc.load_gather(w_v, [po + c])
                    plsc.store_scatter(out_v, [obase + (EMB + 7 + c)], vo)
                for j in range(7):
                    vs = plsc.load_gather(x_v, [sbase + (1 + j)])
                    plsc.store_scatter(out_v, [obase + (EMB + j)], vs)
                    vs2 = plsc.load_gather(x_v, [sbase + (9 + j)])
                    plsc.store_scatter(out_v, [obase + (2 * EMB + 7 + j)], vs2)
                return 0

            lax.fori_loop(0, CHUNK // LANES, group_body, 0)
            pltpu.sync_copy(out_v, out_hbm.at[pl.ds(tb * OUT_D, CHUNK * OUT_D)])
            return 0

        lax.fori_loop(0, n_chunks, chunk_body, 0)

    return k(x_flat, w_flat)


def kernel(x, W):
    b, t, f = x.shape
    num_tokens = b * t
    out_flat = _sc_embed(x.reshape(-1), W.reshape(-1), num_tokens=num_tokens)
    return out_flat.reshape(b, t, OUT_D)
